# Initial kernel scaffold; baseline (speedup 1.0000x reference)
import functools

import jax
import jax.numpy as jnp
from jax import lax
from jax.experimental import pallas as pl
from jax.experimental.pallas import tpu as pltpu
from jax.experimental.pallas import tpu_sc as plsc

N = 10000
E = 320000
G = 64
CIN = 128
HID = 128

NC = 2
NS = 16
NW = NC * NS
EPW = E // NW
CH = 80
NCHUNK = EPW // CH
RPT = N // NS

BLK = 2000
NBLK = N // BLK

_MESH = plsc.VectorSubcoreMesh(core_axis_name="c", subcore_axis_name="s")


@functools.partial(
    pl.kernel,
    out_type=jax.ShapeDtypeStruct((NC, N, 16), jnp.float32),
    mesh=_MESH,
    scratch_types=[
        pltpu.VMEM((CH,), jnp.int32),
        pltpu.VMEM((CH, 16), jnp.float32),
        pltpu.VMEM_SHARED((N, 16), jnp.float32),
    ],
)
def _sc_deg(dst_hbm, zero16_hbm, out_hbm, dst_v, ones_v, deg_sh):
    cid = lax.axis_index("c")
    sid = lax.axis_index("s")
    w = sid * NC + cid
    r0 = sid * RPT
    pltpu.sync_copy(zero16_hbm.at[pl.ds(r0, RPT)], deg_sh.at[pl.ds(r0, RPT)])
    one = jnp.ones((16,), jnp.float32)
    for r in range(CH):
        ones_v[r, :] = one
    plsc.subcore_barrier()
    e0 = w * EPW

    def body(k, _):
        base = e0 + k * CH
        pltpu.sync_copy(dst_hbm.at[pl.ds(base, CH)], dst_v)
        pltpu.sync_copy(ones_v, deg_sh.at[dst_v], add=True)
        return 0

    lax.fori_loop(0, NCHUNK, body, 0)
    plsc.subcore_barrier()
    pltpu.sync_copy(deg_sh.at[pl.ds(r0, RPT)], out_hbm.at[cid, pl.ds(r0, RPT)])


@functools.partial(
    pl.kernel,
    out_type=jax.ShapeDtypeStruct((NC, N, HID), jnp.float32),
    mesh=_MESH,
    scratch_types=[
        pltpu.VMEM((CH,), jnp.int32),
        pltpu.VMEM((CH,), jnp.int32),
        pltpu.VMEM((CH, HID), jnp.float32),
        pltpu.VMEM_SHARED((N, HID), jnp.float32),
        pltpu.SemaphoreType.DMA,
    ],
)
def _sc_hop(p_hbm, src_hbm, dst_hbm, zero_hbm, out_hbm,
            src_v, dst_v, rows_v, q_sh, sem):
    cid = lax.axis_index("c")
    sid = lax.axis_index("s")
    w = sid * NC + cid
    r0 = sid * RPT

    @pl.when(cid == 0)
    def _():
        pltpu.sync_copy(p_hbm.at[pl.ds(r0, RPT)], q_sh.at[pl.ds(r0, RPT)])

    @pl.when(cid == 1)
    def _():
        pltpu.sync_copy(zero_hbm.at[pl.ds(r0, RPT)], q_sh.at[pl.ds(r0, RPT)])

    plsc.subcore_barrier()
    e0 = w * EPW

    def body(k, _):
        base = e0 + k * CH
        pltpu.sync_copy(src_hbm.at[pl.ds(base, CH)], src_v)
        pltpu.sync_copy(dst_hbm.at[pl.ds(base, CH)], dst_v)
        pltpu.async_copy(p_hbm.at[src_v], rows_v, sem).wait()
        pltpu.sync_copy(rows_v, q_sh.at[dst_v], add=True)
        return 0

    lax.fori_loop(0, NCHUNK, body, 0)
    plsc.subcore_barrier()
    pltpu.sync_copy(q_sh.at[pl.ds(r0, RPT)], out_hbm.at[cid, pl.ds(r0, RPT)])


def _tc_prep_body(degA_ref, degB_ref, x_ref, dinv_ref, dinv2_ref, p1_ref):
    deg = degA_ref[:, 0:1] + degB_ref[:, 0:1] + 1.0
    dinv = lax.rsqrt(deg)
    dinv_ref[...] = dinv
    dinv2_ref[...] = 1.0 / deg
    p1_ref[...] = x_ref[...] * dinv


def _tc_prep(degA, degB, x):
    return pl.pallas_call(
        _tc_prep_body,
        grid=(NBLK,),
        in_specs=[
            pl.BlockSpec((BLK, 16), lambda i: (i, 0)),
            pl.BlockSpec((BLK, 16), lambda i: (i, 0)),
            pl.BlockSpec((BLK, CIN), lambda i: (i, 0)),
        ],
        out_specs=[
            pl.BlockSpec((BLK, 1), lambda i: (i, 0)),
            pl.BlockSpec((BLK, 1), lambda i: (i, 0)),
            pl.BlockSpec((BLK, CIN), lambda i: (i, 0)),
        ],
        out_shape=[
            jax.ShapeDtypeStruct((N, 1), jnp.float32),
            jax.ShapeDtypeStruct((N, 1), jnp.float32),
            jax.ShapeDtypeStruct((N, CIN), jnp.float32),
        ],
    )(degA, degB, x)


def _tc_comb_body(qA_ref, qB_ref, dinv2_ref, out_ref):
    out_ref[...] = (qA_ref[...] + qB_ref[...]) * dinv2_ref[...]


def _tc_comb(qA, qB, dinv2):
    return pl.pallas_call(
        _tc_comb_body,
        grid=(NBLK,),
        in_specs=[
            pl.BlockSpec((BLK, HID), lambda i: (i, 0)),
            pl.BlockSpec((BLK, HID), lambda i: (i, 0)),
            pl.BlockSpec((BLK, 1), lambda i: (i, 0)),
        ],
        out_specs=pl.BlockSpec((BLK, HID), lambda i: (i, 0)),
        out_shape=jax.ShapeDtypeStruct((N, HID), jnp.float32),
    )(qA, qB, dinv2)


def _tc_tail_body(qA_ref, qB_ref, dinv_ref, batch_ref,
                  W_ref, b_ref, W1_ref, b1_ref, W2_ref, b2_ref,
                  out_ref, sums_ref, cnt_ref):
    i = pl.program_id(0)
    h = (qA_ref[...] + qB_ref[...]) * dinv_ref[...]
    hW = lax.dot_general(h, W_ref[...], (((1,), (0,)), ((), ())),
                         preferred_element_type=jnp.float32) + b_ref[...]
    hW = jnp.maximum(hW, 0.0)
    bt = batch_ref[0]
    gi = lax.broadcasted_iota(jnp.int32, (G, 1), 0)
    ohT = (bt == gi).astype(jnp.float32)
    part = lax.dot_general(ohT, hW, (((1,), (0,)), ((), ())),
                           preferred_element_type=jnp.float32)
    cntp = jnp.sum(ohT, axis=1, keepdims=True)

    @pl.when(i == 0)
    def _():
        sums_ref[...] = part
        cnt_ref[...] = cntp

    @pl.when(i > 0)
    def _():
        sums_ref[...] = sums_ref[...] + part
        cnt_ref[...] = cnt_ref[...] + cntp

    @pl.when(i == NBLK - 1)
    def _():
        havg = sums_ref[...] / jnp.maximum(cnt_ref[...], 1.0)
        z = lax.dot_general(havg, W1_ref[...], (((1,), (0,)), ((), ())),
                            preferred_element_type=jnp.float32) + b1_ref[...]
        z = jnp.maximum(z, 0.0)
        logits = lax.dot_general(z, W2_ref[...], (((1,), (0,)), ((), ())),
                                 preferred_element_type=jnp.float32) + b2_ref[...]
        out_ref[...] = logits


def _tc_tail(qA, qB, dinv, batch3, W, b2d, W1, b12d, W2, b22d):
    return pl.pallas_call(
        _tc_tail_body,
        grid=(NBLK,),
        in_specs=[
            pl.BlockSpec((BLK, HID), lambda i: (i, 0)),
            pl.BlockSpec((BLK, HID), lambda i: (i, 0)),
            pl.BlockSpec((BLK, 1), lambda i: (i, 0)),
            pl.BlockSpec((1, 1, BLK), lambda i: (i, 0, 0)),
            pl.BlockSpec((HID, HID), lambda i: (0, 0)),
            pl.BlockSpec((1, HID), lambda i: (0, 0)),
            pl.BlockSpec((HID, HID // 2), lambda i: (0, 0)),
            pl.BlockSpec((1, HID // 2), lambda i: (0, 0)),
            pl.BlockSpec((HID // 2, 10), lambda i: (0, 0)),
            pl.BlockSpec((1, 10), lambda i: (0, 0)),
        ],
        out_specs=pl.BlockSpec((G, 10), lambda i: (0, 0)),
        out_shape=jax.ShapeDtypeStruct((G, 10), jnp.float32),
        scratch_shapes=[
            pltpu.VMEM((G, HID), jnp.float32),
            pltpu.VMEM((G, 1), jnp.float32),
        ],
    )(qA, qB, dinv, batch3, W, b2d, W1, b12d, W2, b22d)


def kernel(x, edge_index, batch, W, b, W1, b1, W2, b2):
    src = edge_index[0]
    dst = edge_index[1]
    zeros16 = jnp.zeros((N, 16), jnp.float32)
    zeros128 = jnp.zeros((N, HID), jnp.float32)

    deg_parts = _sc_deg(dst, zeros16)
    dinv, dinv2, p1 = _tc_prep(deg_parts[0], deg_parts[1], x)

    q1 = _sc_hop(p1, src, dst, zeros128)
    p2 = _tc_comb(q1[0], q1[1], dinv2)
    q2 = _sc_hop(p2, src, dst, zeros128)
    p3 = _tc_comb(q2[0], q2[1], dinv2)
    q3 = _sc_hop(p3, src, dst, zeros128)

    batch3 = batch.reshape(NBLK, 1, BLK)
    logits = _tc_tail(q3[0], q3[1], dinv, batch3,
                      W, b.reshape(1, HID), W1, b1.reshape(1, HID // 2),
                      W2, b2.reshape(1, 10))
    return logits


# speedup vs baseline: 11.0440x; 11.0440x over previous
import functools

import jax
import jax.numpy as jnp
from jax import lax
from jax.experimental import pallas as pl
from jax.experimental.pallas import tpu as pltpu
from jax.experimental.pallas import tpu_sc as plsc

N = 10000
E = 320000
G = 64
CIN = 128
HID = 128

NC = 2
NS = 16
NW = NC * NS
EPW = E // NW
CH = 80
NCHUNK = EPW // CH
RPT = 624
TAIL = N - RPT * NS

BLK = 2000
NBLK = N // BLK

_MESH = plsc.VectorSubcoreMesh(core_axis_name="c", subcore_axis_name="s",
                               num_cores=NC, num_subcores=NS)


@functools.partial(
    pl.kernel,
    out_type=jax.ShapeDtypeStruct((NC, N, 16), jnp.float32),
    mesh=_MESH,
    scratch_types=[
        pltpu.VMEM((CH,), jnp.int32),
        pltpu.VMEM((CH, 16), jnp.float32),
        pltpu.VMEM_SHARED((N, 16), jnp.float32),
    ],
)
def _sc_deg(dst_hbm, zero16_hbm, out_hbm, dst_v, ones_v, deg_sh):
    cid = lax.axis_index("c")
    sid = lax.axis_index("s")
    w = sid * NC + cid
    r0 = sid * RPT
    pltpu.sync_copy(zero16_hbm.at[pl.ds(r0, RPT)], deg_sh.at[pl.ds(r0, RPT)])

    @pl.when(sid == NS - 1)
    def _():
        pltpu.sync_copy(zero16_hbm.at[pl.ds(RPT * NS, TAIL)],
                        deg_sh.at[pl.ds(RPT * NS, TAIL)])

    one = jnp.ones((16,), jnp.float32)
    for r in range(CH):
        ones_v[r, :] = one
    plsc.subcore_barrier()
    e0 = w * EPW

    def body(k, _):
        base = e0 + k * CH
        pltpu.sync_copy(dst_hbm.at[pl.ds(base, CH)], dst_v)
        pltpu.sync_copy(ones_v, deg_sh.at[dst_v], add=True)
        return 0

    lax.fori_loop(0, NCHUNK, body, 0)
    plsc.subcore_barrier()
    pltpu.sync_copy(deg_sh.at[pl.ds(r0, RPT)], out_hbm.at[cid, pl.ds(r0, RPT)])

    @pl.when(sid == NS - 1)
    def _():
        pltpu.sync_copy(deg_sh.at[pl.ds(RPT * NS, TAIL)],
                        out_hbm.at[cid, pl.ds(RPT * NS, TAIL)])


@functools.partial(
    pl.kernel,
    out_type=jax.ShapeDtypeStruct((NC, N, HID), jnp.float32),
    mesh=_MESH,
    scratch_types=[
        pltpu.VMEM((CH,), jnp.int32),
        pltpu.VMEM((CH,), jnp.int32),
        pltpu.VMEM((CH, HID), jnp.float32),
        pltpu.VMEM_SHARED((N, HID), jnp.float32),
        pltpu.SemaphoreType.DMA,
    ],
)
def _sc_hop(p_hbm, src_hbm, dst_hbm, zero_hbm, out_hbm,
            src_v, dst_v, rows_v, q_sh, sem):
    cid = lax.axis_index("c")
    sid = lax.axis_index("s")
    w = sid * NC + cid
    r0 = sid * RPT

    @pl.when(cid == 0)
    def _():
        pltpu.sync_copy(p_hbm.at[pl.ds(r0, RPT)], q_sh.at[pl.ds(r0, RPT)])

        @pl.when(sid == NS - 1)
        def _():
            pltpu.sync_copy(p_hbm.at[pl.ds(RPT * NS, TAIL)],
                            q_sh.at[pl.ds(RPT * NS, TAIL)])

    @pl.when(cid == 1)
    def _():
        pltpu.sync_copy(zero_hbm.at[pl.ds(r0, RPT)], q_sh.at[pl.ds(r0, RPT)])

        @pl.when(sid == NS - 1)
        def _():
            pltpu.sync_copy(zero_hbm.at[pl.ds(RPT * NS, TAIL)],
                            q_sh.at[pl.ds(RPT * NS, TAIL)])

    plsc.subcore_barrier()
    e0 = w * EPW

    def body(k, _):
        base = e0 + k * CH
        pltpu.sync_copy(src_hbm.at[pl.ds(base, CH)], src_v)
        pltpu.sync_copy(dst_hbm.at[pl.ds(base, CH)], dst_v)
        pltpu.async_copy(p_hbm.at[src_v], rows_v, sem).wait()
        pltpu.sync_copy(rows_v, q_sh.at[dst_v], add=True)
        return 0

    lax.fori_loop(0, NCHUNK, body, 0)
    plsc.subcore_barrier()
    pltpu.sync_copy(q_sh.at[pl.ds(r0, RPT)], out_hbm.at[cid, pl.ds(r0, RPT)])

    @pl.when(sid == NS - 1)
    def _():
        pltpu.sync_copy(q_sh.at[pl.ds(RPT * NS, TAIL)],
                        out_hbm.at[cid, pl.ds(RPT * NS, TAIL)])


def _tc_prep_body(degA_ref, degB_ref, x_ref, dinv_ref, dinv2_ref, p1_ref):
    deg = degA_ref[:, 0:1] + degB_ref[:, 0:1] + 1.0
    dinv = lax.rsqrt(deg)
    dinv_ref[...] = dinv
    dinv2_ref[...] = 1.0 / deg
    p1_ref[...] = x_ref[...] * dinv


def _tc_prep(degA, degB, x):
    return pl.pallas_call(
        _tc_prep_body,
        grid=(NBLK,),
        in_specs=[
            pl.BlockSpec((BLK, 16), lambda i: (i, 0)),
            pl.BlockSpec((BLK, 16), lambda i: (i, 0)),
            pl.BlockSpec((BLK, CIN), lambda i: (i, 0)),
        ],
        out_specs=[
            pl.BlockSpec((BLK, 1), lambda i: (i, 0)),
            pl.BlockSpec((BLK, 1), lambda i: (i, 0)),
            pl.BlockSpec((BLK, CIN), lambda i: (i, 0)),
        ],
        out_shape=[
            jax.ShapeDtypeStruct((N, 1), jnp.float32),
            jax.ShapeDtypeStruct((N, 1), jnp.float32),
            jax.ShapeDtypeStruct((N, CIN), jnp.float32),
        ],
    )(degA, degB, x)


def _tc_comb_body(qA_ref, qB_ref, dinv2_ref, out_ref):
    out_ref[...] = (qA_ref[...] + qB_ref[...]) * dinv2_ref[...]


def _tc_comb(qA, qB, dinv2):
    return pl.pallas_call(
        _tc_comb_body,
        grid=(NBLK,),
        in_specs=[
            pl.BlockSpec((BLK, HID), lambda i: (i, 0)),
            pl.BlockSpec((BLK, HID), lambda i: (i, 0)),
            pl.BlockSpec((BLK, 1), lambda i: (i, 0)),
        ],
        out_specs=pl.BlockSpec((BLK, HID), lambda i: (i, 0)),
        out_shape=jax.ShapeDtypeStruct((N, HID), jnp.float32),
    )(qA, qB, dinv2)


def _tc_tail_body(qA_ref, qB_ref, dinv_ref, batch_ref,
                  W_ref, b_ref, W1_ref, b1_ref, W2_ref, b2_ref,
                  out_ref, sums_ref, cnt_ref):
    i = pl.program_id(0)
    h = (qA_ref[...] + qB_ref[...]) * dinv_ref[...]
    hW = lax.dot_general(h, W_ref[...], (((1,), (0,)), ((), ())),
                         preferred_element_type=jnp.float32) + b_ref[...]
    hW = jnp.maximum(hW, 0.0)
    bt = batch_ref[0]
    gi = lax.broadcasted_iota(jnp.int32, (G, 1), 0)
    ohT = (bt == gi).astype(jnp.float32)
    part = lax.dot_general(ohT, hW, (((1,), (0,)), ((), ())),
                           preferred_element_type=jnp.float32)
    cntp = jnp.sum(ohT, axis=1, keepdims=True)

    @pl.when(i == 0)
    def _():
        sums_ref[...] = part
        cnt_ref[...] = cntp

    @pl.when(i > 0)
    def _():
        sums_ref[...] = sums_ref[...] + part
        cnt_ref[...] = cnt_ref[...] + cntp

    @pl.when(i == NBLK - 1)
    def _():
        havg = sums_ref[...] / jnp.maximum(cnt_ref[...], 1.0)
        z = lax.dot_general(havg, W1_ref[...], (((1,), (0,)), ((), ())),
                            preferred_element_type=jnp.float32) + b1_ref[...]
        z = jnp.maximum(z, 0.0)
        logits = lax.dot_general(z, W2_ref[...], (((1,), (0,)), ((), ())),
                                 preferred_element_type=jnp.float32) + b2_ref[...]
        out_ref[...] = logits


def _tc_tail(qA, qB, dinv, batch3, W, b2d, W1, b12d, W2, b22d):
    return pl.pallas_call(
        _tc_tail_body,
        grid=(NBLK,),
        in_specs=[
            pl.BlockSpec((BLK, HID), lambda i: (i, 0)),
            pl.BlockSpec((BLK, HID), lambda i: (i, 0)),
            pl.BlockSpec((BLK, 1), lambda i: (i, 0)),
            pl.BlockSpec((1, 1, BLK), lambda i: (i, 0, 0)),
            pl.BlockSpec((HID, HID), lambda i: (0, 0)),
            pl.BlockSpec((1, HID), lambda i: (0, 0)),
            pl.BlockSpec((HID, HID // 2), lambda i: (0, 0)),
            pl.BlockSpec((1, HID // 2), lambda i: (0, 0)),
            pl.BlockSpec((HID // 2, 10), lambda i: (0, 0)),
            pl.BlockSpec((1, 10), lambda i: (0, 0)),
        ],
        out_specs=pl.BlockSpec((G, 10), lambda i: (0, 0)),
        out_shape=jax.ShapeDtypeStruct((G, 10), jnp.float32),
        scratch_shapes=[
            pltpu.VMEM((G, HID), jnp.float32),
            pltpu.VMEM((G, 1), jnp.float32),
        ],
    )(qA, qB, dinv, batch3, W, b2d, W1, b12d, W2, b22d)


def kernel(x, edge_index, batch, W, b, W1, b1, W2, b2):
    src = edge_index[0]
    dst = edge_index[1]
    zeros16 = jnp.zeros((N, 16), jnp.float32)
    zeros128 = jnp.zeros((N, HID), jnp.float32)

    deg_parts = _sc_deg(dst, zeros16)
    dinv, dinv2, p1 = _tc_prep(deg_parts[0], deg_parts[1], x)

    q1 = _sc_hop(p1, src, dst, zeros128)
    p2 = _tc_comb(q1[0], q1[1], dinv2)
    q2 = _sc_hop(p2, src, dst, zeros128)
    p3 = _tc_comb(q2[0], q2[1], dinv2)
    q3 = _sc_hop(p3, src, dst, zeros128)

    batch3 = batch.reshape(NBLK, 1, BLK)
    logits = _tc_tail(q3[0], q3[1], dinv, batch3,
                      W, b.reshape(1, HID), W1, b1.reshape(1, HID // 2),
                      W2, b2.reshape(1, 10))
    return logits


# speedup vs baseline: 25.4818x; 2.3073x over previous
import functools

import jax
import jax.numpy as jnp
from jax import lax
from jax.experimental import pallas as pl
from jax.experimental.pallas import tpu as pltpu
from jax.experimental.pallas import tpu_sc as plsc

N = 10000
E = 320000
G = 64
CIN = 128
HID = 128

NC = 2
NS = 16
NW = NC * NS
EPW = E // NW
CH = 80
NCHUNK = EPW // CH
RPT = 624
TAIL = N - RPT * NS

BLK = 2000
NBLK = N // BLK

_MESH = plsc.VectorSubcoreMesh(core_axis_name="c", subcore_axis_name="s",
                               num_cores=NC, num_subcores=NS)


@functools.partial(
    pl.kernel,
    out_type=jax.ShapeDtypeStruct((NC, N, 16), jnp.float32),
    mesh=_MESH,
    scratch_types=[
        pltpu.VMEM((NCHUNK, CH), jnp.int32),
        pltpu.VMEM((CH, 16), jnp.float32),
        pltpu.VMEM_SHARED((N, 16), jnp.float32),
        pltpu.SemaphoreType.DMA,
    ],
)
def _sc_deg(dst3_hbm, zero16_hbm, out_hbm, dsts_v, ones_v, deg_sh, sem):
    cid = lax.axis_index("c")
    sid = lax.axis_index("s")
    w = sid * NC + cid
    r0 = sid * RPT
    pltpu.sync_copy(dst3_hbm.at[w], dsts_v)
    pltpu.sync_copy(zero16_hbm.at[pl.ds(r0, RPT)], deg_sh.at[pl.ds(r0, RPT)])

    @pl.when(sid == NS - 1)
    def _():
        pltpu.sync_copy(zero16_hbm.at[pl.ds(RPT * NS, TAIL)],
                        deg_sh.at[pl.ds(RPT * NS, TAIL)])

    one = jnp.ones((16,), jnp.float32)
    for r in range(CH):
        ones_v[r, :] = one
    plsc.subcore_barrier()

    def scat(c):
        return pltpu.async_copy(ones_v, deg_sh.at[dsts_v.at[c]], sem, add=True)

    def wait_scat():
        pltpu.make_async_copy(ones_v, deg_sh.at[dsts_v.at[0]], sem).wait()

    scat(0)
    scat(1)

    def body(k, _):
        scat(k + 2)
        wait_scat()
        return 0

    lax.fori_loop(0, NCHUNK - 2, body, 0)
    wait_scat()
    wait_scat()
    plsc.subcore_barrier()
    pltpu.sync_copy(deg_sh.at[pl.ds(r0, RPT)], out_hbm.at[cid, pl.ds(r0, RPT)])

    @pl.when(sid == NS - 1)
    def _():
        pltpu.sync_copy(deg_sh.at[pl.ds(RPT * NS, TAIL)],
                        out_hbm.at[cid, pl.ds(RPT * NS, TAIL)])


@functools.partial(
    pl.kernel,
    out_type=jax.ShapeDtypeStruct((NC, N, HID), jnp.float32),
    mesh=_MESH,
    scratch_types=[
        pltpu.VMEM((NCHUNK, CH), jnp.int32),
        pltpu.VMEM((2, CH), jnp.int32),
        pltpu.VMEM((2, CH, HID), jnp.float32),
        pltpu.VMEM_SHARED((N, HID), jnp.float32),
        pltpu.SemaphoreType.DMA,
        pltpu.SemaphoreType.DMA,
    ],
)
def _sc_hop(p_hbm, src3_hbm, dst3_hbm, zero_hbm, out_hbm,
            srcs_v, dsts2_v, rows_v, q_sh, sem_g, sem_i):
    cid = lax.axis_index("c")
    sid = lax.axis_index("s")
    w = sid * NC + cid
    r0 = sid * RPT
    pltpu.sync_copy(src3_hbm.at[w], srcs_v)

    @pl.when(cid == 0)
    def _():
        pltpu.sync_copy(p_hbm.at[pl.ds(r0, RPT)], q_sh.at[pl.ds(r0, RPT)])

        @pl.when(sid == NS - 1)
        def _():
            pltpu.sync_copy(p_hbm.at[pl.ds(RPT * NS, TAIL)],
                            q_sh.at[pl.ds(RPT * NS, TAIL)])

    @pl.when(cid == 1)
    def _():
        pltpu.sync_copy(zero_hbm.at[pl.ds(r0, RPT)], q_sh.at[pl.ds(r0, RPT)])

        @pl.when(sid == NS - 1)
        def _():
            pltpu.sync_copy(zero_hbm.at[pl.ds(RPT * NS, TAIL)],
                            q_sh.at[pl.ds(RPT * NS, TAIL)])

    plsc.subcore_barrier()

    def gather(c, b):
        pltpu.async_copy(p_hbm.at[srcs_v.at[c]], rows_v.at[b], sem_g)

    def wait_gather(b):
        pltpu.make_async_copy(p_hbm.at[srcs_v.at[0]], rows_v.at[b], sem_g).wait()

    def load_dst(c, b):
        pltpu.async_copy(dst3_hbm.at[w, c], dsts2_v.at[b], sem_i)

    def wait_dst(b):
        pltpu.make_async_copy(dst3_hbm.at[w, 0], dsts2_v.at[b], sem_i).wait()

    def scatter(c, b):
        pltpu.sync_copy(rows_v.at[b], q_sh.at[dsts2_v.at[b]], add=True)

    load_dst(0, 0)
    gather(0, 0)

    def substep(c, b):
        gather(c + 1, 1 - b)
        load_dst(c + 1, 1 - b)
        wait_gather(b)
        wait_dst(b)
        scatter(c, b)

    def body(j, _):
        substep(2 * j, 0)
        substep(2 * j + 1, 1)
        return 0

    lax.fori_loop(0, (NCHUNK - 1) // 2, body, 0)
    wait_gather(0)
    wait_dst(0)
    scatter(NCHUNK - 1, 0)
    plsc.subcore_barrier()
    pltpu.sync_copy(q_sh.at[pl.ds(r0, RPT)], out_hbm.at[cid, pl.ds(r0, RPT)])

    @pl.when(sid == NS - 1)
    def _():
        pltpu.sync_copy(q_sh.at[pl.ds(RPT * NS, TAIL)],
                        out_hbm.at[cid, pl.ds(RPT * NS, TAIL)])


def _tc_prep_body(degA_ref, degB_ref, x_ref, dinv_ref, dinv2_ref, p1_ref):
    deg = degA_ref[:, 0:1] + degB_ref[:, 0:1] + 1.0
    dinv = lax.rsqrt(deg)
    dinv_ref[...] = dinv
    dinv2_ref[...] = 1.0 / deg
    p1_ref[...] = x_ref[...] * dinv


def _tc_prep(degA, degB, x):
    return pl.pallas_call(
        _tc_prep_body,
        grid=(NBLK,),
        in_specs=[
            pl.BlockSpec((BLK, 16), lambda i: (i, 0)),
            pl.BlockSpec((BLK, 16), lambda i: (i, 0)),
            pl.BlockSpec((BLK, CIN), lambda i: (i, 0)),
        ],
        out_specs=[
            pl.BlockSpec((BLK, 1), lambda i: (i, 0)),
            pl.BlockSpec((BLK, 1), lambda i: (i, 0)),
            pl.BlockSpec((BLK, CIN), lambda i: (i, 0)),
        ],
        out_shape=[
            jax.ShapeDtypeStruct((N, 1), jnp.float32),
            jax.ShapeDtypeStruct((N, 1), jnp.float32),
            jax.ShapeDtypeStruct((N, CIN), jnp.float32),
        ],
    )(degA, degB, x)


def _tc_comb_body(qA_ref, qB_ref, dinv2_ref, out_ref):
    out_ref[...] = (qA_ref[...] + qB_ref[...]) * dinv2_ref[...]


def _tc_comb(qA, qB, dinv2):
    return pl.pallas_call(
        _tc_comb_body,
        grid=(NBLK,),
        in_specs=[
            pl.BlockSpec((BLK, HID), lambda i: (i, 0)),
            pl.BlockSpec((BLK, HID), lambda i: (i, 0)),
            pl.BlockSpec((BLK, 1), lambda i: (i, 0)),
        ],
        out_specs=pl.BlockSpec((BLK, HID), lambda i: (i, 0)),
        out_shape=jax.ShapeDtypeStruct((N, HID), jnp.float32),
    )(qA, qB, dinv2)


def _tc_tail_body(qA_ref, qB_ref, dinv_ref, batch_ref,
                  W_ref, b_ref, W1_ref, b1_ref, W2_ref, b2_ref,
                  out_ref, sums_ref, cnt_ref):
    i = pl.program_id(0)
    h = (qA_ref[...] + qB_ref[...]) * dinv_ref[...]
    hW = lax.dot_general(h, W_ref[...], (((1,), (0,)), ((), ())),
                         preferred_element_type=jnp.float32) + b_ref[...]
    hW = jnp.maximum(hW, 0.0)
    bt = batch_ref[0]
    gi = lax.broadcasted_iota(jnp.int32, (G, 1), 0)
    ohT = (bt == gi).astype(jnp.float32)
    part = lax.dot_general(ohT, hW, (((1,), (0,)), ((), ())),
                           preferred_element_type=jnp.float32)
    cntp = jnp.sum(ohT, axis=1, keepdims=True)

    @pl.when(i == 0)
    def _():
        sums_ref[...] = part
        cnt_ref[...] = cntp

    @pl.when(i > 0)
    def _():
        sums_ref[...] = sums_ref[...] + part
        cnt_ref[...] = cnt_ref[...] + cntp

    @pl.when(i == NBLK - 1)
    def _():
        havg = sums_ref[...] / jnp.maximum(cnt_ref[...], 1.0)
        z = lax.dot_general(havg, W1_ref[...], (((1,), (0,)), ((), ())),
                            preferred_element_type=jnp.float32) + b1_ref[...]
        z = jnp.maximum(z, 0.0)
        logits = lax.dot_general(z, W2_ref[...], (((1,), (0,)), ((), ())),
                                 preferred_element_type=jnp.float32) + b2_ref[...]
        out_ref[...] = logits


def _tc_tail(qA, qB, dinv, batch3, W, b2d, W1, b12d, W2, b22d):
    return pl.pallas_call(
        _tc_tail_body,
        grid=(NBLK,),
        in_specs=[
            pl.BlockSpec((BLK, HID), lambda i: (i, 0)),
            pl.BlockSpec((BLK, HID), lambda i: (i, 0)),
            pl.BlockSpec((BLK, 1), lambda i: (i, 0)),
            pl.BlockSpec((1, 1, BLK), lambda i: (i, 0, 0)),
            pl.BlockSpec((HID, HID), lambda i: (0, 0)),
            pl.BlockSpec((1, HID), lambda i: (0, 0)),
            pl.BlockSpec((HID, HID // 2), lambda i: (0, 0)),
            pl.BlockSpec((1, HID // 2), lambda i: (0, 0)),
            pl.BlockSpec((HID // 2, 10), lambda i: (0, 0)),
            pl.BlockSpec((1, 10), lambda i: (0, 0)),
        ],
        out_specs=pl.BlockSpec((G, 10), lambda i: (0, 0)),
        out_shape=jax.ShapeDtypeStruct((G, 10), jnp.float32),
        scratch_shapes=[
            pltpu.VMEM((G, HID), jnp.float32),
            pltpu.VMEM((G, 1), jnp.float32),
        ],
    )(qA, qB, dinv, batch3, W, b2d, W1, b12d, W2, b22d)


def kernel(x, edge_index, batch, W, b, W1, b1, W2, b2):
    src3 = edge_index[0].reshape(NW, NCHUNK, CH)
    dst3 = edge_index[1].reshape(NW, NCHUNK, CH)
    zeros16 = jnp.zeros((N, 16), jnp.float32)
    zeros128 = jnp.zeros((N, HID), jnp.float32)

    deg_parts = _sc_deg(dst3, zeros16)
    dinv, dinv2, p1 = _tc_prep(deg_parts[0], deg_parts[1], x)

    q1 = _sc_hop(p1, src3, dst3, zeros128)
    p2 = _tc_comb(q1[0], q1[1], dinv2)
    q2 = _sc_hop(p2, src3, dst3, zeros128)
    p3 = _tc_comb(q2[0], q2[1], dinv2)
    q3 = _sc_hop(p3, src3, dst3, zeros128)

    batch3 = batch.reshape(NBLK, 1, BLK)
    logits = _tc_tail(q3[0], q3[1], dinv, batch3,
                      W, b.reshape(1, HID), W1, b1.reshape(1, HID // 2),
                      W2, b2.reshape(1, 10))
    return logits
